# baseline (speedup 1.0000x reference)
import functools

import jax
import jax.numpy as jnp
from jax import lax
from jax.experimental import pallas as pl
from jax.experimental.pallas import tpu as pltpu
from jax.experimental.pallas import tpu_sc as plsc

N = 10000
E = 320000
D = 128

N_PAD = 10240
NC = 2
NT = 16
ROWS_PER_TILE = N_PAD // NT
EDGES_PER_CORE = E // NC
EDGES_PER_TILE = EDGES_PER_CORE // NT
CH = 125
CHUNKS = EDGES_PER_TILE // CH

_MESH = plsc.VectorSubcoreMesh(core_axis_name="c", subcore_axis_name="s")


def _deg_body(edge4_hbm, out_hbm, dst_i, ones_v, zero_v, cnt_sh, sem):
    c = lax.axis_index("c")
    s = lax.axis_index("s")

    for i in range(ones_v.shape[0] // 16):
        ones_v[pl.ds(i * 16, 16)] = jnp.ones((16,), jnp.float32)
    npt = N_PAD // NT
    for i in range(npt // 16):
        zero_v[pl.ds(i * 16, 16)] = jnp.zeros((16,), jnp.float32)
    pltpu.sync_copy(zero_v, cnt_sh.at[pl.ds(s * npt, npt)])
    pltpu.sync_copy(edge4_hbm.at[1, c * NT + s], dst_i)
    plsc.subcore_barrier()

    DEPTH = 16

    def fire(j, carry):
        @pl.when(j >= DEPTH)
        def _():
            pltpu.make_async_copy(ones_v.at[pl.ds(0, CH)], cnt_sh.at[dst_i.at[j - DEPTH]],
                                  sem).wait()

        pltpu.async_copy(ones_v.at[pl.ds(0, CH)], cnt_sh.at[dst_i.at[j]], sem, add=True)
        return carry

    lax.fori_loop(0, CHUNKS, fire, 0)

    def drain(j, carry):
        pltpu.make_async_copy(ones_v.at[pl.ds(0, CH)], cnt_sh.at[dst_i.at[j]], sem).wait()
        return carry

    lax.fori_loop(CHUNKS - DEPTH, CHUNKS, drain, 0)
    plsc.subcore_barrier()
    pltpu.sync_copy(cnt_sh.at[pl.ds(s * npt, npt)],
                    out_hbm.at[c, pl.ds(s * npt, npt)])


_deg_kernel = functools.partial(
    pl.kernel,
    out_type=jax.ShapeDtypeStruct((NC, N_PAD), jnp.float32),
    mesh=_MESH,
    scratch_types=[
        pltpu.VMEM((CHUNKS, CH), jnp.int32),
        pltpu.VMEM((-(-CH // 16) * 16,), jnp.float32),
        pltpu.VMEM((N_PAD // NT,), jnp.float32),
        pltpu.VMEM_SHARED((N_PAD,), jnp.float32),
        pltpu.SemaphoreType.DMA,
    ],
)(_deg_body)


_R = 1000


def _matmul_body(z_ref, w_ref, cnt_ref, hs_ref, dis_ref):
    deg = cnt_ref[:, 0:1] + cnt_ref[:, 1:2] + 1.0
    dis = lax.rsqrt(deg)
    h = jnp.dot(z_ref[...], w_ref[...], preferred_element_type=jnp.float32)
    hs_ref[...] = h * dis
    dis_ref[...] = dis


def _tc_matmul_scale(z, W, cnt_t):
    return pl.pallas_call(
        _matmul_body,
        grid=(N // _R,),
        in_specs=[
            pl.BlockSpec((_R, D), lambda i: (i, 0)),
            pl.BlockSpec((D, D), lambda i: (0, 0)),
            pl.BlockSpec((_R, 2), lambda i: (i, 0)),
        ],
        out_specs=[
            pl.BlockSpec((_R, D), lambda i: (i, 0)),
            pl.BlockSpec((_R, 1), lambda i: (i, 0)),
        ],
        out_shape=[
            jax.ShapeDtypeStruct((N, D), jnp.float32),
            jax.ShapeDtypeStruct((N, 1), jnp.float32),
        ],
    )(z, W, cnt_t)


def _scatter_body(hs_hbm, edge4_hbm, out_hbm, src_i, dstb,
                  rows0, rows1, acc_sh, sem0, sem1):
    c = lax.axis_index("c")
    s = lax.axis_index("s")

    for i in range(CH):
        for j in range(D // 16):
            rows0[i, pl.ds(j * 16, 16)] = jnp.zeros((16,), jnp.float32)
            rows1[i, pl.ds(j * 16, 16)] = jnp.zeros((16,), jnp.float32)
    for k in range(ROWS_PER_TILE // CH):
        pltpu.sync_copy(rows0, acc_sh.at[pl.ds(s * ROWS_PER_TILE + k * CH, CH)])
    _zrem = ROWS_PER_TILE % CH
    if _zrem:
        pltpu.sync_copy(
            rows0.at[pl.ds(0, _zrem)],
            acc_sh.at[pl.ds(s * ROWS_PER_TILE + (ROWS_PER_TILE // CH) * CH,
                            _zrem)])

    wid = c * NT + s
    pltpu.sync_copy(edge4_hbm.at[0, wid], src_i)
    plsc.subcore_barrier()

    pltpu.async_copy(hs_hbm.at[src_i.at[0]], rows0, sem0)
    pltpu.sync_copy(edge4_hbm.at[1, wid, 0], dstb.at[0])

    def step(jj, carry):
        j0 = 2 * jj
        pltpu.async_copy(hs_hbm.at[src_i.at[j0 + 1]], rows1, sem1)
        pltpu.sync_copy(edge4_hbm.at[1, wid, j0 + 1], dstb.at[1])
        pltpu.make_async_copy(hs_hbm.at[src_i.at[j0]], rows0, sem0).wait()
        pltpu.sync_copy(rows0, acc_sh.at[dstb.at[0]], add=True)

        @pl.when(j0 + 2 < CHUNKS)
        def _():
            pltpu.async_copy(hs_hbm.at[src_i.at[j0 + 2]], rows0, sem0)
            pltpu.sync_copy(edge4_hbm.at[1, wid, j0 + 2], dstb.at[0])

        pltpu.make_async_copy(hs_hbm.at[src_i.at[j0 + 1]], rows1, sem1).wait()
        pltpu.sync_copy(rows1, acc_sh.at[dstb.at[1]], add=True)
        return carry

    lax.fori_loop(0, CHUNKS // 2, step, 0)
    if CHUNKS % 2 == 1:
        pltpu.make_async_copy(hs_hbm.at[src_i.at[CHUNKS - 1]], rows0, sem0).wait()
        pltpu.sync_copy(rows0, acc_sh.at[dstb.at[0]], add=True)

    plsc.subcore_barrier()
    pltpu.sync_copy(acc_sh.at[pl.ds(s * ROWS_PER_TILE, ROWS_PER_TILE)],
                    out_hbm.at[c, pl.ds(s * ROWS_PER_TILE, ROWS_PER_TILE)])


_scatter_kernel = functools.partial(
    pl.kernel,
    out_type=jax.ShapeDtypeStruct((NC, N_PAD, D), jnp.float32),
    mesh=_MESH,
    scratch_types=[
        pltpu.VMEM((CHUNKS, CH), jnp.int32),
        pltpu.VMEM((2, CH), jnp.int32),
        pltpu.VMEM((CH, D), jnp.float32),
        pltpu.VMEM((CH, D), jnp.float32),
        pltpu.VMEM_SHARED((N_PAD, D), jnp.float32),
        pltpu.SemaphoreType.DMA,
        pltpu.SemaphoreType.DMA,
    ],
)(_scatter_body)


def _finish_body(acc_ref, hs_ref, dis_ref, b_ref, out_ref):
    agg = acc_ref[0] + acc_ref[1] + hs_ref[...]
    out_ref[...] = jnp.maximum(agg * dis_ref[...] + b_ref[...], 0.0)


def _tc_finish(acc, hs, dis, b2):
    return pl.pallas_call(
        _finish_body,
        grid=(N // _R,),
        in_specs=[
            pl.BlockSpec((NC, _R, D), lambda i: (0, i, 0)),
            pl.BlockSpec((_R, D), lambda i: (i, 0)),
            pl.BlockSpec((_R, 1), lambda i: (i, 0)),
            pl.BlockSpec((1, D), lambda i: (0, 0)),
        ],
        out_specs=pl.BlockSpec((_R, D), lambda i: (i, 0)),
        out_shape=jax.ShapeDtypeStruct((N, D), jnp.float32),
    )(acc, hs, dis, b2)


def kernel(z, edge_index, W, b):
    edge4 = edge_index.astype(jnp.int32).reshape(2, NC * NT, CHUNKS, CH)
    counts = _deg_kernel(edge4)
    cnt_t = counts.T
    hs, dis = _tc_matmul_scale(z, W, cnt_t)
    acc = _scatter_kernel(hs, edge4)
    return _tc_finish(acc, hs, dis, b[None, :])


# speedup vs baseline: 1.0259x; 1.0259x over previous
import functools

import jax
import jax.numpy as jnp
from jax import lax
from jax.experimental import pallas as pl
from jax.experimental.pallas import tpu as pltpu
from jax.experimental.pallas import tpu_sc as plsc

N = 10000
E = 320000
D = 128

N_PAD = 10240
NC = 2
NT = 16
ROWS_PER_TILE = N_PAD // NT
EDGES_PER_CORE = E // NC
EDGES_PER_TILE = EDGES_PER_CORE // NT
CH = 125
CHUNKS = EDGES_PER_TILE // CH

_MESH = plsc.VectorSubcoreMesh(core_axis_name="c", subcore_axis_name="s")


def _deg_body(edge4_hbm, out_hbm, dst_i, ones_v, zero_v, cnt_sh, sem):
    c = lax.axis_index("c")
    s = lax.axis_index("s")

    for i in range(ones_v.shape[0] // 16):
        ones_v[pl.ds(i * 16, 16)] = jnp.ones((16,), jnp.float32)
    npt = N_PAD // NT
    for i in range(npt // 16):
        zero_v[pl.ds(i * 16, 16)] = jnp.zeros((16,), jnp.float32)
    pltpu.sync_copy(zero_v, cnt_sh.at[pl.ds(s * npt, npt)])
    pltpu.sync_copy(edge4_hbm.at[1, c * NT + s], dst_i)
    plsc.subcore_barrier()

    DEPTH = 16

    def fire(j, carry):
        @pl.when(j >= DEPTH)
        def _():
            pltpu.make_async_copy(ones_v.at[pl.ds(0, CH)], cnt_sh.at[dst_i.at[j - DEPTH]],
                                  sem).wait()

        pltpu.async_copy(ones_v.at[pl.ds(0, CH)], cnt_sh.at[dst_i.at[j]], sem, add=True)
        return carry

    lax.fori_loop(0, CHUNKS, fire, 0)

    def drain(j, carry):
        pltpu.make_async_copy(ones_v.at[pl.ds(0, CH)], cnt_sh.at[dst_i.at[j]], sem).wait()
        return carry

    lax.fori_loop(CHUNKS - DEPTH, CHUNKS, drain, 0)
    plsc.subcore_barrier()
    pltpu.sync_copy(cnt_sh.at[pl.ds(s * npt, npt)],
                    out_hbm.at[c, pl.ds(s * npt, npt)])


_deg_kernel = functools.partial(
    pl.kernel,
    out_type=jax.ShapeDtypeStruct((NC, N_PAD), jnp.float32),
    mesh=_MESH,
    scratch_types=[
        pltpu.VMEM((CHUNKS, CH), jnp.int32),
        pltpu.VMEM((-(-CH // 16) * 16,), jnp.float32),
        pltpu.VMEM((N_PAD // NT,), jnp.float32),
        pltpu.VMEM_SHARED((N_PAD,), jnp.float32),
        pltpu.SemaphoreType.DMA,
    ],
)(_deg_body)


_R = 2000


def _matmul_body(z_ref, w_ref, cnt_ref, hs_ref, dis_ref):
    deg = cnt_ref[:, 0:1] + cnt_ref[:, 1:2] + 1.0
    dis = lax.rsqrt(deg)
    h = jnp.dot(z_ref[...], w_ref[...], preferred_element_type=jnp.float32)
    hs_ref[...] = h * dis
    dis_ref[...] = dis


def _tc_matmul_scale(z, W, cnt_t):
    return pl.pallas_call(
        _matmul_body,
        grid=(N // _R,),
        in_specs=[
            pl.BlockSpec((_R, D), lambda i: (i, 0)),
            pl.BlockSpec((D, D), lambda i: (0, 0)),
            pl.BlockSpec((_R, 2), lambda i: (i, 0)),
        ],
        out_specs=[
            pl.BlockSpec((_R, D), lambda i: (i, 0)),
            pl.BlockSpec((_R, 1), lambda i: (i, 0)),
        ],
        out_shape=[
            jax.ShapeDtypeStruct((N, D), jnp.float32),
            jax.ShapeDtypeStruct((N, 1), jnp.float32),
        ],
    )(z, W, cnt_t)


def _scatter_body(hs_hbm, edge4_hbm, out_hbm, src_i, dstb,
                  rows0, rows1, acc_sh, sem0, sem1):
    c = lax.axis_index("c")
    s = lax.axis_index("s")

    for i in range(CH):
        for j in range(D // 16):
            rows0[i, pl.ds(j * 16, 16)] = jnp.zeros((16,), jnp.float32)
            rows1[i, pl.ds(j * 16, 16)] = jnp.zeros((16,), jnp.float32)
    for k in range(ROWS_PER_TILE // CH):
        pltpu.sync_copy(rows0, acc_sh.at[pl.ds(s * ROWS_PER_TILE + k * CH, CH)])
    _zrem = ROWS_PER_TILE % CH
    if _zrem:
        pltpu.sync_copy(
            rows0.at[pl.ds(0, _zrem)],
            acc_sh.at[pl.ds(s * ROWS_PER_TILE + (ROWS_PER_TILE // CH) * CH,
                            _zrem)])

    wid = c * NT + s
    pltpu.sync_copy(edge4_hbm.at[0, wid], src_i)
    plsc.subcore_barrier()

    pltpu.async_copy(hs_hbm.at[src_i.at[0]], rows0, sem0)
    pltpu.sync_copy(edge4_hbm.at[1, wid, 0], dstb.at[0])

    def step(jj, carry):
        j0 = 2 * jj
        pltpu.async_copy(hs_hbm.at[src_i.at[j0 + 1]], rows1, sem1)
        pltpu.sync_copy(edge4_hbm.at[1, wid, j0 + 1], dstb.at[1])
        pltpu.make_async_copy(hs_hbm.at[src_i.at[j0]], rows0, sem0).wait()
        pltpu.sync_copy(rows0, acc_sh.at[dstb.at[0]], add=True)

        @pl.when(j0 + 2 < CHUNKS)
        def _():
            pltpu.async_copy(hs_hbm.at[src_i.at[j0 + 2]], rows0, sem0)
            pltpu.sync_copy(edge4_hbm.at[1, wid, j0 + 2], dstb.at[0])

        pltpu.make_async_copy(hs_hbm.at[src_i.at[j0 + 1]], rows1, sem1).wait()
        pltpu.sync_copy(rows1, acc_sh.at[dstb.at[1]], add=True)
        return carry

    lax.fori_loop(0, CHUNKS // 2, step, 0)
    if CHUNKS % 2 == 1:
        pltpu.make_async_copy(hs_hbm.at[src_i.at[CHUNKS - 1]], rows0, sem0).wait()
        pltpu.sync_copy(rows0, acc_sh.at[dstb.at[0]], add=True)

    plsc.subcore_barrier()
    pltpu.sync_copy(acc_sh.at[pl.ds(s * ROWS_PER_TILE, ROWS_PER_TILE)],
                    out_hbm.at[c, pl.ds(s * ROWS_PER_TILE, ROWS_PER_TILE)])


_scatter_kernel = functools.partial(
    pl.kernel,
    out_type=jax.ShapeDtypeStruct((NC, N_PAD, D), jnp.float32),
    mesh=_MESH,
    scratch_types=[
        pltpu.VMEM((CHUNKS, CH), jnp.int32),
        pltpu.VMEM((2, CH), jnp.int32),
        pltpu.VMEM((CH, D), jnp.float32),
        pltpu.VMEM((CH, D), jnp.float32),
        pltpu.VMEM_SHARED((N_PAD, D), jnp.float32),
        pltpu.SemaphoreType.DMA,
        pltpu.SemaphoreType.DMA,
    ],
)(_scatter_body)


def _finish_body(acc_ref, hs_ref, dis_ref, b_ref, out_ref):
    agg = acc_ref[0] + acc_ref[1] + hs_ref[...]
    out_ref[...] = jnp.maximum(agg * dis_ref[...] + b_ref[...], 0.0)


def _tc_finish(acc, hs, dis, b2):
    return pl.pallas_call(
        _finish_body,
        grid=(N // _R,),
        in_specs=[
            pl.BlockSpec((NC, _R, D), lambda i: (0, i, 0)),
            pl.BlockSpec((_R, D), lambda i: (i, 0)),
            pl.BlockSpec((_R, 1), lambda i: (i, 0)),
            pl.BlockSpec((1, D), lambda i: (0, 0)),
        ],
        out_specs=pl.BlockSpec((_R, D), lambda i: (i, 0)),
        out_shape=jax.ShapeDtypeStruct((N, D), jnp.float32),
    )(acc, hs, dis, b2)


def kernel(z, edge_index, W, b):
    edge4 = edge_index.astype(jnp.int32).reshape(2, NC * NT, CHUNKS, CH)
    counts = _deg_kernel(edge4)
    cnt_t = counts.T
    hs, dis = _tc_matmul_scale(z, W, cnt_t)
    acc = _scatter_kernel(hs, edge4)
    return _tc_finish(acc, hs, dis, b[None, :])
